# baseline (speedup 1.0000x reference)
import jax
import jax.numpy as jnp
from jax.experimental import pallas as pl
from jax.experimental.pallas import tpu as pltpu


def _body(ar_ref, gate_ref, x_ref, emb_ref, o_ref):
    bi = pl.program_id(0)
    ti = pl.program_id(1)
    w = ar_ref[bi, 0]
    h = ar_ref[bi, 1]
    g = jnp.tanh(gate_ref[0])
    scale = jnp.where(ti < w * h, g, jnp.zeros_like(g))
    o_ref[...] = x_ref[...] + emb_ref[...] * scale


def kernel(x, ar, embedding, gate):
    b, t, n, w = x.shape

    def x_map(bi, ti, ar_ref, gate_ref):
        return (bi, ti, 0, 0)

    def emb_map(bi, ti, ar_ref, gate_ref):
        h = ar_ref[bi, 1]
        return (ti // h, ti % h, 0, 0)

    grid_spec = pltpu.PrefetchScalarGridSpec(
        num_scalar_prefetch=2,
        grid=(b, t),
        in_specs=[
            pl.BlockSpec((1, 1, n, w), x_map),
            pl.BlockSpec((1, 1, 1, w), emb_map),
        ],
        out_specs=pl.BlockSpec((1, 1, n, w), x_map),
    )
    return pl.pallas_call(
        _body,
        grid_spec=grid_spec,
        out_shape=jax.ShapeDtypeStruct(x.shape, x.dtype),
    )(ar, gate, x, embedding)
